# Initial kernel scaffold; baseline (speedup 1.0000x reference)
import jax
import jax.numpy as jnp
from jax.experimental import pallas as pl


def kernel(q, k, v, pos_enc, kq_map):
    raise NotImplementedError("write your pallas kernel here")


# speedup vs baseline: 1.5647x; 1.5647x over previous
import functools

import jax
import jax.numpy as jnp
from jax.experimental import pallas as pl
from jax.experimental.pallas import tpu as pltpu

R = 10240


def _matmul_body(a_ref, b_ref, o_ref):
    o_ref[...] = jax.lax.dot_general(
        a_ref[...], b_ref[...], (((1,), (1,)), ((), ())),
        preferred_element_type=jnp.float32)


def _scores(q_pad, kp):
    BM = BN = 512
    grid = (R // BM, R // BN)
    return pl.pallas_call(
        _matmul_body,
        grid=grid,
        in_specs=[
            pl.BlockSpec((BM, q_pad.shape[1]), lambda i, j: (i, 0)),
            pl.BlockSpec((BN, kp.shape[1]), lambda i, j: (j, 0)),
        ],
        out_specs=pl.BlockSpec((BM, BN), lambda i, j: (i, j)),
        out_shape=jax.ShapeDtypeStruct((R, R), jnp.float32),
    )(q_pad, kp)


def kernel(q, k, v, pos_enc, kq_map):
    N, C = q.shape
    K = pos_enc.shape[0]
    q_pad = jnp.zeros((R, C), q.dtype).at[:N].set(q)
    kp = jnp.zeros((R, C), k.dtype).at[:N].set(k).at[N:N + K].set(pos_enc)
    S = _scores(q_pad, kp).reshape(-1)

    key_idx = kq_map[0] // K
    kernel_idx = kq_map[0] - key_idx * K
    out_idx = kq_map[1]
    scale = 1.0 / (C ** 0.5)
    logits = (S[out_idx * R + key_idx] + S[out_idx * R + N + kernel_idx]) * scale
    e = jnp.exp(logits)
    denom = jax.ops.segment_sum(e, out_idx, num_segments=N)
    attn = e / denom[out_idx]
    out = jax.ops.segment_sum(attn[:, None] * jnp.take(v, key_idx, axis=0),
                              out_idx, num_segments=N)
    return out


# speedup vs baseline: 1.5651x; 1.0003x over previous
import functools

import jax
import jax.numpy as jnp
from jax import lax
from jax.experimental import pallas as pl
from jax.experimental.pallas import tpu as pltpu
from jax.experimental.pallas import tpu_sc as plsc

R = 10240
NV = 10000
KV = 27
ND = 10112
MP = 270336
TP = MP // 32
CH = 128
SCALE = 1.0 / 16.0


def _matmul_body(a_ref, b_ref, o_ref):
    o_ref[...] = jax.lax.dot_general(
        a_ref[...], b_ref[...], (((1,), (1,)), ((), ())),
        preferred_element_type=jnp.float32)


def _scores(q_pad, kp):
    BM = BN = 512
    grid = (R // BM, R // BN)
    return pl.pallas_call(
        _matmul_body,
        grid=grid,
        in_specs=[
            pl.BlockSpec((BM, q_pad.shape[1]), lambda i, j: (i, 0)),
            pl.BlockSpec((BN, kp.shape[1]), lambda i, j: (j, 0)),
        ],
        out_specs=pl.BlockSpec((BM, BN), lambda i, j: (i, j)),
        out_shape=jax.ShapeDtypeStruct((R, R), jnp.float32),
    )(q_pad, kp)


def _k2_body(sf, kq0, kq1, e_out, dpart,
             kq0_v, kq1_v, idx1, idx2, s1, s2, e_v, den_v, sem1, sem2):
    c = lax.axis_index("c")
    s = lax.axis_index("s")
    wid = s * 2 + c

    zero16 = jnp.zeros((16,), jnp.float32)

    def zbody(i, _):
        den_v[pl.ds(i * 16, 16)] = zero16
        return 0

    lax.fori_loop(0, ND // 16, zbody, 0)

    def chunk(ci, _):
        base = wid * TP + ci * CH
        pltpu.sync_copy(kq0.at[pl.ds(base, CH)], kq0_v)
        pltpu.sync_copy(kq1.at[pl.ds(base, CH)], kq1_v)
        for g in range(CH // 16):
            a = kq0_v[pl.ds(g * 16, 16)]
            o = kq1_v[pl.ds(g * 16, 16)]
            kkey = a // KV
            kid = a - kkey * KV
            idx1[pl.ds(g * 16, 16)] = o * R + kkey
            idx2[pl.ds(g * 16, 16)] = o * R + (NV + kid)
        cp1 = pltpu.async_copy(sf.at[idx1], s1, sem1)
        cp2 = pltpu.async_copy(sf.at[idx2], s2, sem2)
        cp1.wait()
        cp2.wait()
        for g in range(CH // 16):
            ev = jnp.exp((s1[pl.ds(g * 16, 16)] + s2[pl.ds(g * 16, 16)]) * SCALE)
            e_v[pl.ds(g * 16, 16)] = ev
            o = kq1_v[pl.ds(g * 16, 16)]
            plsc.addupdate_scatter(den_v, [o], ev)
        pltpu.sync_copy(e_v, e_out.at[pl.ds(base, CH)])
        return 0

    lax.fori_loop(0, TP // CH, chunk, 0)
    pltpu.sync_copy(den_v, dpart.at[wid])


def _k2(sf, kq0, kq1):
    mesh = plsc.VectorSubcoreMesh(core_axis_name="c", subcore_axis_name="s")
    f = pl.kernel(
        _k2_body,
        compiler_params=pltpu.CompilerParams(needs_layout_passes=False),
        out_type=[
            jax.ShapeDtypeStruct((MP,), jnp.float32),
            jax.ShapeDtypeStruct((32, ND), jnp.float32),
        ],
        mesh=mesh,
        scratch_types=[
            pltpu.VMEM((CH,), jnp.int32),
            pltpu.VMEM((CH,), jnp.int32),
            pltpu.VMEM((CH,), jnp.int32),
            pltpu.VMEM((CH,), jnp.int32),
            pltpu.VMEM((CH,), jnp.float32),
            pltpu.VMEM((CH,), jnp.float32),
            pltpu.VMEM((CH,), jnp.float32),
            pltpu.VMEM((ND,), jnp.float32),
            pltpu.SemaphoreType.DMA,
            pltpu.SemaphoreType.DMA,
        ],
    )
    return f(sf, kq0, kq1)


def _k3_body(dp_ref, o_ref):
    o_ref[...] = 1.0 / jnp.sum(dp_ref[...], axis=0, keepdims=True)


def _k3(dpart):
    return pl.pallas_call(
        _k3_body,
        out_shape=jax.ShapeDtypeStruct((1, ND), jnp.float32),
    )(dpart)


def kernel(q, k, v, pos_enc, kq_map):
    N, C = q.shape
    K = pos_enc.shape[0]
    q_pad = jnp.zeros((R, C), q.dtype).at[:N].set(q)
    kp = jnp.zeros((R, C), k.dtype).at[:N].set(k).at[N:N + K].set(pos_enc)
    S = _scores(q_pad, kp).reshape(-1)

    pad = MP - kq_map.shape[1]
    kq0 = jnp.concatenate([kq_map[0], jnp.zeros((pad,), kq_map.dtype)])
    kq1 = jnp.concatenate([kq_map[1], jnp.full((pad,), NV, kq_map.dtype)])

    e, dpart = _k2(S, kq0, kq1)
    invd = _k3(dpart).reshape(-1)

    key_idx = kq_map[0] // K
    out_idx = kq_map[1]
    attn = e[:kq_map.shape[1]] * invd[out_idx]
    out = jax.ops.segment_sum(attn[:, None] * jnp.take(v, key_idx, axis=0),
                              out_idx, num_segments=N)
    return out


# speedup vs baseline: 5.1329x; 3.2795x over previous
import functools

import jax
import jax.numpy as jnp
from jax import lax
from jax.experimental import pallas as pl
from jax.experimental.pallas import tpu as pltpu
from jax.experimental.pallas import tpu_sc as plsc

R = 10240
NV = 10000
KV = 27
ND = 10112
MP = 270336
TP = MP // 32
CH = 128
SCALE = 1.0 / 16.0


def _matmul_body(a_ref, b_ref, o_ref):
    o_ref[...] = jax.lax.dot_general(
        a_ref[...], b_ref[...], (((1,), (1,)), ((), ())),
        preferred_element_type=jnp.float32)


def _scores(q_pad, kp):
    BM = BN = 512
    grid = (R // BM, R // BN)
    return pl.pallas_call(
        _matmul_body,
        grid=grid,
        in_specs=[
            pl.BlockSpec((BM, q_pad.shape[1]), lambda i, j: (i, 0)),
            pl.BlockSpec((BN, kp.shape[1]), lambda i, j: (j, 0)),
        ],
        out_specs=pl.BlockSpec((BM, BN), lambda i, j: (i, j)),
        out_shape=jax.ShapeDtypeStruct((R, R), jnp.float32),
    )(q_pad, kp)


def _k2_body(sf, kq0, kq1, e_out, dpart,
             kq0_v, kq1_v, idx1, idx2, s1, s2, e_v, den_v, sem1, sem2):
    c = lax.axis_index("c")
    s = lax.axis_index("s")
    wid = s * 2 + c

    zero16 = jnp.zeros((16,), jnp.float32)

    def zbody(i, _):
        den_v[pl.ds(i * 16, 16)] = zero16
        return 0

    lax.fori_loop(0, ND // 16, zbody, 0)

    def chunk(ci, _):
        base = wid * TP + ci * CH
        pltpu.sync_copy(kq0.at[pl.ds(base, CH)], kq0_v)
        pltpu.sync_copy(kq1.at[pl.ds(base, CH)], kq1_v)
        for g in range(CH // 16):
            a = kq0_v[pl.ds(g * 16, 16)]
            o = kq1_v[pl.ds(g * 16, 16)]
            kkey = a // KV
            kid = a - kkey * KV
            idx1[pl.ds(g * 16, 16)] = o * R + kkey
            idx2[pl.ds(g * 16, 16)] = o * R + (NV + kid)
        cp1 = pltpu.async_copy(sf.at[idx1], s1, sem1)
        cp2 = pltpu.async_copy(sf.at[idx2], s2, sem2)
        cp1.wait()
        cp2.wait()
        for g in range(CH // 16):
            ev = jnp.exp((s1[pl.ds(g * 16, 16)] + s2[pl.ds(g * 16, 16)]) * SCALE)
            e_v[pl.ds(g * 16, 16)] = ev
            o = kq1_v[pl.ds(g * 16, 16)]
            plsc.addupdate_scatter(den_v, [o], ev)
        pltpu.sync_copy(e_v, e_out.at[pl.ds(base, CH)])
        return 0

    lax.fori_loop(0, TP // CH, chunk, 0)
    pltpu.sync_copy(den_v, dpart.at[wid])


def _k2(sf, kq0, kq1):
    mesh = plsc.VectorSubcoreMesh(core_axis_name="c", subcore_axis_name="s")
    f = pl.kernel(
        _k2_body,
        compiler_params=pltpu.CompilerParams(needs_layout_passes=False),
        out_type=[
            jax.ShapeDtypeStruct((MP,), jnp.float32),
            jax.ShapeDtypeStruct((32, ND), jnp.float32),
        ],
        mesh=mesh,
        scratch_types=[
            pltpu.VMEM((CH,), jnp.int32),
            pltpu.VMEM((CH,), jnp.int32),
            pltpu.VMEM((CH,), jnp.int32),
            pltpu.VMEM((CH,), jnp.int32),
            pltpu.VMEM((CH,), jnp.float32),
            pltpu.VMEM((CH,), jnp.float32),
            pltpu.VMEM((CH,), jnp.float32),
            pltpu.VMEM((ND,), jnp.float32),
            pltpu.SemaphoreType.DMA,
            pltpu.SemaphoreType.DMA,
        ],
    )
    return f(sf, kq0, kq1)


TPW = MP // 16
NSTR = ND // 16


def _k4_body(vcat, e_in, invd, kq0, kq1, zer, out_hbm,
             invd_v, kq0_v, kq1_v, vidx, e_v, attn_v, rows_v, out_sp, sem):
    c = lax.axis_index("c")
    s = lax.axis_index("s")
    pltpu.sync_copy(invd, invd_v)
    pltpu.sync_copy(zer, out_sp.at[pl.ds(s * NSTR, NSTR)])
    plsc.subcore_barrier()
    coff = c * NV

    def chunk(ci, _):
        base = s * TPW + ci * CH
        pltpu.sync_copy(kq0.at[pl.ds(base, CH)], kq0_v)
        pltpu.sync_copy(kq1.at[pl.ds(base, CH)], kq1_v)
        pltpu.sync_copy(e_in.at[pl.ds(base, CH)], e_v)
        for g in range(CH // 16):
            a = kq0_v[pl.ds(g * 16, 16)]
            o = kq1_v[pl.ds(g * 16, 16)]
            vidx[pl.ds(g * 16, 16)] = a // KV + coff
            d = plsc.load_gather(invd_v, [o])
            attn_v[pl.ds(g * 16, 16)] = e_v[pl.ds(g * 16, 16)] * d
        pltpu.async_copy(vcat.at[vidx], rows_v, sem).wait()

        def scale(p, _):
            a16 = plsc.load_gather(attn_v, [jnp.zeros((16,), jnp.int32) + p])
            for j in range(8):
                rows_v[p, pl.ds(j * 16, 16)] = rows_v[p, pl.ds(j * 16, 16)] * a16
            return 0

        lax.fori_loop(0, CH, scale, 0)
        pltpu.sync_copy(rows_v, out_sp.at[kq1_v], add=True)
        return 0

    lax.fori_loop(0, TPW // CH, chunk, 0)
    plsc.subcore_barrier()
    pltpu.sync_copy(out_sp.at[pl.ds(s * NSTR, NSTR)],
                    out_hbm.at[pl.ds(c * ND + s * NSTR, NSTR)])


def _k4(vcat, e, invd, kq0, kq1, zer):
    mesh = plsc.VectorSubcoreMesh(core_axis_name="c", subcore_axis_name="s")
    f = pl.kernel(
        _k4_body,
        compiler_params=pltpu.CompilerParams(needs_layout_passes=False),
        out_type=jax.ShapeDtypeStruct((2 * ND, 128), jnp.float32),
        mesh=mesh,
        scratch_types=[
            pltpu.VMEM((ND,), jnp.float32),
            pltpu.VMEM((CH,), jnp.int32),
            pltpu.VMEM((CH,), jnp.int32),
            pltpu.VMEM((CH,), jnp.int32),
            pltpu.VMEM((CH,), jnp.float32),
            pltpu.VMEM((CH,), jnp.float32),
            pltpu.VMEM((CH, 128), jnp.float32),
            pltpu.VMEM_SHARED((ND, 128), jnp.float32),
            pltpu.SemaphoreType.DMA,
        ],
    )
    return f(vcat, e, invd, kq0, kq1, zer)


def _k3_body(dp_ref, o_ref):
    o_ref[...] = 1.0 / jnp.sum(dp_ref[...], axis=0, keepdims=True)


def _k3(dpart):
    return pl.pallas_call(
        _k3_body,
        out_shape=jax.ShapeDtypeStruct((1, ND), jnp.float32),
    )(dpart)


def kernel(q, k, v, pos_enc, kq_map):
    N, C = q.shape
    K = pos_enc.shape[0]
    q_pad = jnp.zeros((R, C), q.dtype).at[:N].set(q)
    kp = jnp.zeros((R, C), k.dtype).at[:N].set(k).at[N:N + K].set(pos_enc)
    S = _scores(q_pad, kp).reshape(-1)

    pad = MP - kq_map.shape[1]
    kq0 = jnp.concatenate([kq_map[0], jnp.zeros((pad,), kq_map.dtype)])
    kq1 = jnp.concatenate([kq_map[1], jnp.full((pad,), NV, kq_map.dtype)])

    e, dpart = _k2(S, kq0, kq1)
    invd = _k3(dpart).reshape(-1)

    vcat = jnp.concatenate([v[:, :128], v[:, 128:]], axis=0)
    zer = jnp.zeros((NSTR, 128), jnp.float32)
    oc = _k4(vcat, e, invd, kq0, kq1, zer)
    return jnp.concatenate([oc[:N], oc[ND:ND + N]], axis=1)


# speedup vs baseline: 6.6294x; 1.2916x over previous
import functools

import jax
import jax.numpy as jnp
from jax import lax
from jax.experimental import pallas as pl
from jax.experimental.pallas import tpu as pltpu
from jax.experimental.pallas import tpu_sc as plsc

R = 10240
NV = 10000
KV = 27
ND = 10112
MP = 270336
TP = MP // 32
CH = 128
SCALE = 1.0 / 16.0


def _matmul_body(a_ref, b_ref, o_ref):
    o_ref[...] = jax.lax.dot_general(
        a_ref[...], b_ref[...], (((1,), (1,)), ((), ())),
        preferred_element_type=jnp.float32)


def _scores(q_pad, kp):
    BM = BN = 512
    grid = (R // BM, R // BN)
    return pl.pallas_call(
        _matmul_body,
        grid=grid,
        in_specs=[
            pl.BlockSpec((BM, q_pad.shape[1]), lambda i, j: (i, 0)),
            pl.BlockSpec((BN, kp.shape[1]), lambda i, j: (j, 0)),
        ],
        out_specs=pl.BlockSpec((BM, BN), lambda i, j: (i, j)),
        out_shape=jax.ShapeDtypeStruct((R, R), jnp.float32),
    )(q_pad, kp)


def _k2_body(sf, kq0, kq1, e_out, dpart,
             kq0_v, kq1_v, idx1, idx2, s1, s2, e_v, den_v, sem1, sem2):
    c = lax.axis_index("c")
    s = lax.axis_index("s")
    wid = s * 2 + c

    zero16 = jnp.zeros((16,), jnp.float32)

    def zbody(i, _):
        den_v[pl.ds(i * 16, 16)] = zero16
        return 0

    lax.fori_loop(0, ND // 16, zbody, 0)

    def chunk(ci, _):
        base = wid * TP + ci * CH
        pltpu.sync_copy(kq0.at[pl.ds(base, CH)], kq0_v)
        pltpu.sync_copy(kq1.at[pl.ds(base, CH)], kq1_v)
        for g in range(CH // 16):
            a = kq0_v[pl.ds(g * 16, 16)]
            o = kq1_v[pl.ds(g * 16, 16)]
            kkey = a // KV
            kid = a - kkey * KV
            idx1[pl.ds(g * 16, 16)] = o * R + kkey
            idx2[pl.ds(g * 16, 16)] = o * R + (NV + kid)
        cp1 = pltpu.async_copy(sf.at[idx1], s1, sem1)
        cp2 = pltpu.async_copy(sf.at[idx2], s2, sem2)
        cp1.wait()
        cp2.wait()
        for g in range(CH // 16):
            ev = jnp.exp((s1[pl.ds(g * 16, 16)] + s2[pl.ds(g * 16, 16)]) * SCALE)
            e_v[pl.ds(g * 16, 16)] = ev
            o = kq1_v[pl.ds(g * 16, 16)]
            plsc.addupdate_scatter(den_v, [o], ev)
        pltpu.sync_copy(e_v, e_out.at[pl.ds(base, CH)])
        return 0

    lax.fori_loop(0, TP // CH, chunk, 0)
    pltpu.sync_copy(den_v, dpart.at[wid])


def _k2(sf, kq0, kq1):
    mesh = plsc.VectorSubcoreMesh(core_axis_name="c", subcore_axis_name="s")
    f = pl.kernel(
        _k2_body,
        compiler_params=pltpu.CompilerParams(needs_layout_passes=False),
        out_type=[
            jax.ShapeDtypeStruct((MP,), jnp.float32),
            jax.ShapeDtypeStruct((32, ND), jnp.float32),
        ],
        mesh=mesh,
        scratch_types=[
            pltpu.VMEM((CH,), jnp.int32),
            pltpu.VMEM((CH,), jnp.int32),
            pltpu.VMEM((CH,), jnp.int32),
            pltpu.VMEM((CH,), jnp.int32),
            pltpu.VMEM((CH,), jnp.float32),
            pltpu.VMEM((CH,), jnp.float32),
            pltpu.VMEM((CH,), jnp.float32),
            pltpu.VMEM((ND,), jnp.float32),
            pltpu.SemaphoreType.DMA,
            pltpu.SemaphoreType.DMA,
        ],
    )
    return f(sf, kq0, kq1)


TPW = MP // 16
NSTR = ND // 16


NCH4 = TPW // CH


def _k4_body(vcat, e_in, invd, kq0, kq1, zer, out_hbm,
             invd_v,
             kq0_a, kq1_a, e_a, vidx_a, oidx_a, attn_a, rows_a,
             kq0_b, kq1_b, e_b, vidx_b, oidx_b, attn_b, rows_b,
             out_sp, sem_a, sem_b, lsem_a, lsem_b):
    c = lax.axis_index("c")
    s = lax.axis_index("s")
    base0 = s * TPW
    coff = c * NV

    def fire_kq(ci, kq0_c, kq1_c, e_c, lsem):
        base = base0 + ci * CH
        pltpu.async_copy(kq0.at[pl.ds(base, CH)], kq0_c, lsem)
        pltpu.async_copy(kq1.at[pl.ds(base, CH)], kq1_c, lsem)
        pltpu.async_copy(e_in.at[pl.ds(base, CH)], e_c, lsem)

    def wait_kq(kq0_c, kq1_c, e_c, lsem):
        pltpu.make_async_copy(kq0.at[pl.ds(0, CH)], kq0_c, lsem).wait()
        pltpu.make_async_copy(kq1.at[pl.ds(0, CH)], kq1_c, lsem).wait()
        pltpu.make_async_copy(e_in.at[pl.ds(0, CH)], e_c, lsem).wait()

    def prep(kq0_c, kq1_c, e_c, vidx_buf, oidx_buf, attn_buf):
        for g in range(CH // 16):
            sl = pl.ds(g * 16, 16)
            a = kq0_c[sl]
            o = kq1_c[sl]
            vidx_buf[sl] = a // KV + coff
            oidx_buf[sl] = o
            d = plsc.load_gather(invd_v, [o])
            attn_buf[sl] = e_c[sl] * d

    def work(rows_buf, oidx_buf, attn_buf):
        def scale(p, _):
            a16 = plsc.load_gather(attn_buf, [jnp.zeros((16,), jnp.int32) + p])
            for j in range(8):
                rows_buf[p, pl.ds(j * 16, 16)] = (
                    rows_buf[p, pl.ds(j * 16, 16)] * a16)
            return 0

        lax.fori_loop(0, CH, scale, 0, unroll=2)
        pltpu.sync_copy(rows_buf, out_sp.at[oidx_buf], add=True)

    fire_kq(0, kq0_a, kq1_a, e_a, lsem_a)
    cpi = pltpu.async_copy(invd, invd_v, sem_a)
    pltpu.sync_copy(zer, out_sp.at[pl.ds(s * NSTR, NSTR)])
    cpi.wait()
    plsc.subcore_barrier()
    wait_kq(kq0_a, kq1_a, e_a, lsem_a)
    prep(kq0_a, kq1_a, e_a, vidx_a, oidx_a, attn_a)
    pltpu.async_copy(vcat.at[vidx_a], rows_a, sem_a)
    fire_kq(1, kq0_b, kq1_b, e_b, lsem_b)

    def pipe(ci2, _):
        o = 2 * ci2 + 1
        wait_kq(kq0_b, kq1_b, e_b, lsem_b)
        prep(kq0_b, kq1_b, e_b, vidx_b, oidx_b, attn_b)
        pltpu.async_copy(vcat.at[vidx_b], rows_b, sem_b)

        @pl.when(o + 1 < NCH4)
        def _():
            fire_kq(o + 1, kq0_a, kq1_a, e_a, lsem_a)

        pltpu.make_async_copy(vcat.at[vidx_a], rows_a, sem_a).wait()
        work(rows_a, oidx_a, attn_a)

        @pl.when(o + 1 < NCH4)
        def _():
            wait_kq(kq0_a, kq1_a, e_a, lsem_a)
            prep(kq0_a, kq1_a, e_a, vidx_a, oidx_a, attn_a)
            pltpu.async_copy(vcat.at[vidx_a], rows_a, sem_a)
            fire_kq(o + 2, kq0_b, kq1_b, e_b, lsem_b)

        pltpu.make_async_copy(vcat.at[vidx_b], rows_b, sem_b).wait()
        work(rows_b, oidx_b, attn_b)
        return 0

    lax.fori_loop(0, NCH4 // 2, pipe, 0)
    plsc.subcore_barrier()
    pltpu.sync_copy(out_sp.at[pl.ds(s * NSTR, NSTR)],
                    out_hbm.at[pl.ds(c * ND + s * NSTR, NSTR)])


def _k4(vcat, e, invd, kq0, kq1, zer):
    mesh = plsc.VectorSubcoreMesh(core_axis_name="c", subcore_axis_name="s")
    buf = [
        pltpu.VMEM((CH,), jnp.int32),
        pltpu.VMEM((CH,), jnp.int32),
        pltpu.VMEM((CH,), jnp.float32),
        pltpu.VMEM((CH,), jnp.int32),
        pltpu.VMEM((CH,), jnp.int32),
        pltpu.VMEM((CH,), jnp.float32),
        pltpu.VMEM((CH, 128), jnp.float32),
    ]
    f = pl.kernel(
        _k4_body,
        compiler_params=pltpu.CompilerParams(needs_layout_passes=False),
        out_type=jax.ShapeDtypeStruct((2 * ND, 128), jnp.float32),
        mesh=mesh,
        scratch_types=(
            [pltpu.VMEM((ND,), jnp.float32)] + buf + buf
            + [
                pltpu.VMEM_SHARED((ND, 128), jnp.float32),
                pltpu.SemaphoreType.DMA,
                pltpu.SemaphoreType.DMA,
                pltpu.SemaphoreType.DMA,
                pltpu.SemaphoreType.DMA,
            ]
        ),
    )
    return f(vcat, e, invd, kq0, kq1, zer)


def _k3_body(dp_ref, o_ref):
    o_ref[...] = 1.0 / jnp.sum(dp_ref[...], axis=0, keepdims=True)


def _k3(dpart):
    return pl.pallas_call(
        _k3_body,
        out_shape=jax.ShapeDtypeStruct((1, ND), jnp.float32),
    )(dpart)


def kernel(q, k, v, pos_enc, kq_map):
    N, C = q.shape
    K = pos_enc.shape[0]
    q_pad = jnp.zeros((R, C), q.dtype).at[:N].set(q)
    kp = jnp.zeros((R, C), k.dtype).at[:N].set(k).at[N:N + K].set(pos_enc)
    S = _scores(q_pad, kp).reshape(-1)

    pad = MP - kq_map.shape[1]
    kq0 = jnp.concatenate([kq_map[0], jnp.zeros((pad,), kq_map.dtype)])
    kq1 = jnp.concatenate([kq_map[1], jnp.full((pad,), NV, kq_map.dtype)])

    e, dpart = _k2(S, kq0, kq1)
    invd = _k3(dpart).reshape(-1)

    vcat = jnp.concatenate([v[:, :128], v[:, 128:]], axis=0)
    zer = jnp.zeros((NSTR, 128), jnp.float32)
    oc = _k4(vcat, e, invd, kq0, kq1, zer)
    return jnp.concatenate([oc[:N], oc[ND:ND + N]], axis=1)


# speedup vs baseline: 12.1002x; 1.8252x over previous
import functools

import jax
import jax.numpy as jnp
from jax import lax
from jax.experimental import pallas as pl
from jax.experimental.pallas import tpu as pltpu
from jax.experimental.pallas import tpu_sc as plsc

NV = 10000
KV = 27
RPAD = 10240
MREAL = NV * KV
MP = 270336
TP = MP // 32
CH = 128
SCALE = 1.0 / 16.0


def _s1_body(a_ref, k_ref, o_ref):
    res = jax.lax.dot_general(
        a_ref[...], k_ref[...], (((1,), (1,)), ((), ())),
        preferred_element_type=jnp.float32)
    res = jnp.concatenate(
        [res, jnp.zeros((res.shape[0], RPAD - NV), jnp.float32)], axis=1)
    o_ref[...] = res.reshape(res.shape[0], RPAD // 128, 128)


def _s1(q, k):
    BM = 400
    return pl.pallas_call(
        _s1_body,
        grid=(NV // BM,),
        in_specs=[
            pl.BlockSpec((BM, 256), lambda i: (i, 0)),
            pl.BlockSpec((NV, 256), lambda i: (0, 0)),
        ],
        out_specs=pl.BlockSpec((BM, RPAD // 128, 128), lambda i: (i, 0, 0)),
        out_shape=jax.ShapeDtypeStruct((NV, RPAD // 128, 128), jnp.float32),
    )(q, k)


def _qp_body(a_ref, p_ref, o_ref):
    res = jax.lax.dot_general(
        a_ref[...], p_ref[...], (((1,), (1,)), ((), ())),
        preferred_element_type=jnp.float32)
    o_ref[...] = jnp.concatenate(
        [res, jnp.zeros((res.shape[0], 128 - KV), jnp.float32)], axis=1)


def _qp(q, pos_enc):
    BM = 2000
    return pl.pallas_call(
        _qp_body,
        grid=(NV // BM,),
        in_specs=[
            pl.BlockSpec((BM, 256), lambda i: (i, 0)),
            pl.BlockSpec((KV, 256), lambda i: (0, 0)),
        ],
        out_specs=pl.BlockSpec((BM, 128), lambda i: (i, 0)),
        out_shape=jax.ShapeDtypeStruct((NV, 128), jnp.float32),
    )(q, pos_enc)


def _vcat_body(v_ref, o_ref):
    o_ref[...] = v_ref[...][None]


def _vcat(v):
    return pl.pallas_call(
        _vcat_body,
        grid=(2,),
        in_specs=[pl.BlockSpec((NV, 128), lambda c: (0, c))],
        out_specs=pl.BlockSpec((1, NV, 128), lambda c: (c, 0, 0)),
        out_shape=jax.ShapeDtypeStruct((2, NV, 128), jnp.float32),
    )(v)


def _k2_body(sf, qpf, kq0, kq1, e_out, dpart,
             kq0_v, kq1_v, idx1, idx2, s1, s2, e_v, den_v, sem1, sem2):
    c = lax.axis_index("c")
    s = lax.axis_index("s")
    wid = s * 2 + c
    iota16 = jax.lax.iota(jnp.int32, 16)

    zero16 = jnp.zeros((16,), jnp.float32)

    def zbody(i, _):
        den_v[pl.ds(i * 16, 16)] = zero16
        return 0

    lax.fori_loop(0, NV // 16, zbody, 0)

    def chunk(ci, _):
        base = wid * TP + ci * CH
        pltpu.sync_copy(kq0.at[pl.ds(base, CH)], kq0_v)
        pltpu.sync_copy(kq1.at[pl.ds(base, CH)], kq1_v)
        for g in range(CH // 16):
            a = kq0_v[pl.ds(g * 16, 16)]
            o = kq1_v[pl.ds(g * 16, 16)]
            kkey = a // KV
            kid = a - kkey * KV
            idx1[pl.ds(g * 16, 16)] = o * RPAD + kkey
            idx2[pl.ds(g * 16, 16)] = o * 128 + kid
        cp1 = pltpu.async_copy(sf.at[idx1], s1, sem1)
        cp2 = pltpu.async_copy(qpf.at[idx2], s2, sem2)
        cp1.wait()
        cp2.wait()
        for g in range(CH // 16):
            ev = jnp.exp((s1[pl.ds(g * 16, 16)] + s2[pl.ds(g * 16, 16)]) * SCALE)
            ev = jnp.where(base + g * 16 + iota16 < MREAL, ev, 0.0)
            e_v[pl.ds(g * 16, 16)] = ev
            o = kq1_v[pl.ds(g * 16, 16)]
            plsc.addupdate_scatter(den_v, [o], ev)
        pltpu.sync_copy(e_v, e_out.at[pl.ds(base, CH)])
        return 0

    lax.fori_loop(0, TP // CH, chunk, 0)
    pltpu.sync_copy(den_v, dpart.at[wid])


def _k2(sf, qpf, kq0, kq1):
    mesh = plsc.VectorSubcoreMesh(core_axis_name="c", subcore_axis_name="s")
    f = pl.kernel(
        _k2_body,
        compiler_params=pltpu.CompilerParams(needs_layout_passes=False),
        out_type=[
            jax.ShapeDtypeStruct((MP,), jnp.float32),
            jax.ShapeDtypeStruct((32, NV), jnp.float32),
        ],
        mesh=mesh,
        scratch_types=[
            pltpu.VMEM((CH,), jnp.int32),
            pltpu.VMEM((CH,), jnp.int32),
            pltpu.VMEM((CH,), jnp.int32),
            pltpu.VMEM((CH,), jnp.int32),
            pltpu.VMEM((CH,), jnp.float32),
            pltpu.VMEM((CH,), jnp.float32),
            pltpu.VMEM((CH,), jnp.float32),
            pltpu.VMEM((NV,), jnp.float32),
            pltpu.SemaphoreType.DMA,
            pltpu.SemaphoreType.DMA,
        ],
    )
    return f(sf, qpf, kq0, kq1)


def _k3_body(dp_ref, o_ref):
    o_ref[...] = 1.0 / jnp.sum(dp_ref[...], axis=0, keepdims=True)


def _k3(dpart):
    return pl.pallas_call(
        _k3_body,
        out_shape=jax.ShapeDtypeStruct((1, NV), jnp.float32),
    )(dpart)


TPW = MP // 16
NSTR = 624
REM = NV - 16 * NSTR
NCH4 = TPW // CH


def _k4_body(vcat, e_in, invd, kq0, kq1, zer, out_hbm,
             invd_v,
             kq0_a, kq1_a, e_a, vidx_a, oidx_a, attn_a, rows_a,
             kq0_b, kq1_b, e_b, vidx_b, oidx_b, attn_b, rows_b,
             out_sp, sem_a, sem_b, lsem_a, lsem_b):
    c = lax.axis_index("c")
    s = lax.axis_index("s")
    base0 = s * TPW
    coff = c * NV

    def fire_kq(ci, kq0_c, kq1_c, e_c, lsem):
        base = base0 + ci * CH
        pltpu.async_copy(kq0.at[pl.ds(base, CH)], kq0_c, lsem)
        pltpu.async_copy(kq1.at[pl.ds(base, CH)], kq1_c, lsem)
        pltpu.async_copy(e_in.at[pl.ds(base, CH)], e_c, lsem)

    def wait_kq(kq0_c, kq1_c, e_c, lsem):
        pltpu.make_async_copy(kq0.at[pl.ds(0, CH)], kq0_c, lsem).wait()
        pltpu.make_async_copy(kq1.at[pl.ds(0, CH)], kq1_c, lsem).wait()
        pltpu.make_async_copy(e_in.at[pl.ds(0, CH)], e_c, lsem).wait()

    def prep(kq0_c, kq1_c, e_c, vidx_buf, oidx_buf, attn_buf):
        for g in range(CH // 16):
            sl = pl.ds(g * 16, 16)
            a = kq0_c[sl]
            o = kq1_c[sl]
            vidx_buf[sl] = a // KV + coff
            oidx_buf[sl] = o
            d = plsc.load_gather(invd_v, [o])
            attn_buf[sl] = e_c[sl] * d

    def work(rows_buf, oidx_buf, attn_buf):
        def scale(p, _):
            a16 = plsc.load_gather(attn_buf, [jnp.zeros((16,), jnp.int32) + p])
            for j in range(8):
                rows_buf[p, pl.ds(j * 16, 16)] = (
                    rows_buf[p, pl.ds(j * 16, 16)] * a16)
            return 0

        lax.fori_loop(0, CH, scale, 0, unroll=2)
        pltpu.sync_copy(rows_buf, out_sp.at[oidx_buf], add=True)

    fire_kq(0, kq0_a, kq1_a, e_a, lsem_a)
    cpi = pltpu.async_copy(invd, invd_v, sem_a)
    pltpu.sync_copy(zer, out_sp.at[pl.ds(s * NSTR, NSTR)])

    @pl.when(s == 0)
    def _():
        pltpu.sync_copy(zer.at[pl.ds(0, REM)],
                        out_sp.at[pl.ds(16 * NSTR, REM)])

    cpi.wait()
    plsc.subcore_barrier()
    wait_kq(kq0_a, kq1_a, e_a, lsem_a)
    prep(kq0_a, kq1_a, e_a, vidx_a, oidx_a, attn_a)
    pltpu.async_copy(vcat.at[vidx_a], rows_a, sem_a)
    fire_kq(1, kq0_b, kq1_b, e_b, lsem_b)

    def pipe(ci2, _):
        o = 2 * ci2 + 1
        wait_kq(kq0_b, kq1_b, e_b, lsem_b)
        prep(kq0_b, kq1_b, e_b, vidx_b, oidx_b, attn_b)
        pltpu.async_copy(vcat.at[vidx_b], rows_b, sem_b)

        @pl.when(o + 1 < NCH4)
        def _():
            fire_kq(o + 1, kq0_a, kq1_a, e_a, lsem_a)

        pltpu.make_async_copy(vcat.at[vidx_a], rows_a, sem_a).wait()
        work(rows_a, oidx_a, attn_a)

        @pl.when(o + 1 < NCH4)
        def _():
            wait_kq(kq0_a, kq1_a, e_a, lsem_a)
            prep(kq0_a, kq1_a, e_a, vidx_a, oidx_a, attn_a)
            pltpu.async_copy(vcat.at[vidx_a], rows_a, sem_a)
            fire_kq(o + 2, kq0_b, kq1_b, e_b, lsem_b)

        pltpu.make_async_copy(vcat.at[vidx_b], rows_b, sem_b).wait()
        work(rows_b, oidx_b, attn_b)
        return 0

    lax.fori_loop(0, NCH4 // 2, pipe, 0)
    plsc.subcore_barrier()
    pltpu.sync_copy(out_sp.at[pl.ds(s * NSTR, NSTR)],
                    out_hbm.at[pl.ds(s * NSTR, NSTR), pl.ds(c * 128, 128)])

    @pl.when(s == 0)
    def _():
        pltpu.sync_copy(out_sp.at[pl.ds(16 * NSTR, REM)],
                        out_hbm.at[pl.ds(16 * NSTR, REM), pl.ds(c * 128, 128)])


def _k4(vcat, e, invd, kq0, kq1, zer):
    mesh = plsc.VectorSubcoreMesh(core_axis_name="c", subcore_axis_name="s")
    buf = [
        pltpu.VMEM((CH,), jnp.int32),
        pltpu.VMEM((CH,), jnp.int32),
        pltpu.VMEM((CH,), jnp.float32),
        pltpu.VMEM((CH,), jnp.int32),
        pltpu.VMEM((CH,), jnp.int32),
        pltpu.VMEM((CH,), jnp.float32),
        pltpu.VMEM((CH, 128), jnp.float32),
    ]
    f = pl.kernel(
        _k4_body,
        compiler_params=pltpu.CompilerParams(needs_layout_passes=False),
        out_type=jax.ShapeDtypeStruct((NV, 256), jnp.float32),
        mesh=mesh,
        scratch_types=(
            [pltpu.VMEM((NV,), jnp.float32)] + buf + buf
            + [
                pltpu.VMEM_SHARED((NV, 128), jnp.float32),
                pltpu.SemaphoreType.DMA,
                pltpu.SemaphoreType.DMA,
                pltpu.SemaphoreType.DMA,
                pltpu.SemaphoreType.DMA,
            ]
        ),
    )
    return f(vcat, e, invd, kq0, kq1, zer)


def kernel(q, k, v, pos_enc, kq_map):
    S1f = _s1(q, k).reshape(-1)
    Qpf = _qp(q, pos_enc).reshape(-1)

    pad = MP - kq_map.shape[1]
    kq0 = jnp.concatenate([kq_map[0], jnp.zeros((pad,), kq_map.dtype)])
    kq1 = jnp.concatenate([kq_map[1], jnp.zeros((pad,), kq_map.dtype)])

    e, dpart = _k2(S1f, Qpf, kq0, kq1)
    invd = _k3(dpart).reshape(-1)

    vcat = _vcat(v).reshape(2 * NV, 128)
    zer = jnp.zeros((NSTR, 128), jnp.float32)
    return _k4(vcat, e, invd, kq0, kq1, zer)


# speedup vs baseline: 12.9689x; 1.0718x over previous
import functools

import jax
import jax.numpy as jnp
from jax import lax
from jax.experimental import pallas as pl
from jax.experimental.pallas import tpu as pltpu
from jax.experimental.pallas import tpu_sc as plsc

NV = 10000
KV = 27
RPAD = 10240
MREAL = NV * KV
MP = 270336
TP = MP // 32
CH = 128
SCALE = 1.0 / 16.0


def _s1_body(a_ref, k_ref, o_ref):
    res = jax.lax.dot_general(
        a_ref[...].astype(jnp.bfloat16), k_ref[...].astype(jnp.bfloat16),
        (((1,), (1,)), ((), ())),
        preferred_element_type=jnp.float32)
    res = jnp.concatenate(
        [res, jnp.zeros((res.shape[0], RPAD - NV), jnp.float32)], axis=1)
    o_ref[...] = res.reshape(res.shape[0], RPAD // 128, 128)


def _s1(q, k):
    BM = 400
    return pl.pallas_call(
        _s1_body,
        grid=(NV // BM,),
        in_specs=[
            pl.BlockSpec((BM, 256), lambda i: (i, 0)),
            pl.BlockSpec((NV, 256), lambda i: (0, 0)),
        ],
        out_specs=pl.BlockSpec((BM, RPAD // 128, 128), lambda i: (i, 0, 0)),
        out_shape=jax.ShapeDtypeStruct((NV, RPAD // 128, 128), jnp.float32),
    )(q, k)


def _qp_body(a_ref, p_ref, o_ref):
    res = jax.lax.dot_general(
        a_ref[...], p_ref[...], (((1,), (1,)), ((), ())),
        preferred_element_type=jnp.float32)
    o_ref[...] = jnp.concatenate(
        [res, jnp.zeros((res.shape[0], 128 - KV), jnp.float32)], axis=1)


def _qp(q, pos_enc):
    BM = 2000
    return pl.pallas_call(
        _qp_body,
        grid=(NV // BM,),
        in_specs=[
            pl.BlockSpec((BM, 256), lambda i: (i, 0)),
            pl.BlockSpec((KV, 256), lambda i: (0, 0)),
        ],
        out_specs=pl.BlockSpec((BM, 128), lambda i: (i, 0)),
        out_shape=jax.ShapeDtypeStruct((NV, 128), jnp.float32),
    )(q, pos_enc)


def _vcat_body(v_ref, o_ref):
    o_ref[...] = v_ref[...][None]


def _vcat(v):
    return pl.pallas_call(
        _vcat_body,
        grid=(2,),
        in_specs=[pl.BlockSpec((NV, 128), lambda c: (0, c))],
        out_specs=pl.BlockSpec((1, NV, 128), lambda c: (c, 0, 0)),
        out_shape=jax.ShapeDtypeStruct((2, NV, 128), jnp.float32),
    )(v)


NCH2 = TP // CH


def _k2_body(sf, qpf, kq0, kq1, e_out, dpart,
             den_v, e_str,
             kq0_a, kq1_a, idx1_a, idx2_a, s1_a, s2_a,
             kq0_b, kq1_b, idx1_b, idx2_b, s1_b, s2_b,
             gsem_a, gsem_b, lsem_a, lsem_b):
    c = lax.axis_index("c")
    s = lax.axis_index("s")
    wid = s * 2 + c
    base0 = wid * TP
    iota16 = jax.lax.iota(jnp.int32, 16)
    zero16 = jnp.zeros((16,), jnp.float32)

    def zbody(i, _):
        den_v[pl.ds(i * 16, 16)] = zero16
        return 0

    def fire_kq(ci, kq0_c, kq1_c, lsem):
        base = base0 + ci * CH
        pltpu.async_copy(kq0.at[pl.ds(base, CH)], kq0_c, lsem)
        pltpu.async_copy(kq1.at[pl.ds(base, CH)], kq1_c, lsem)

    def wait_kq(kq0_c, kq1_c, lsem):
        pltpu.make_async_copy(kq0.at[pl.ds(0, CH)], kq0_c, lsem).wait()
        pltpu.make_async_copy(kq1.at[pl.ds(0, CH)], kq1_c, lsem).wait()

    def prep(kq0_c, kq1_c, idx1_c, idx2_c):
        for g in range(CH // 16):
            sl = pl.ds(g * 16, 16)
            a = kq0_c[sl]
            o = kq1_c[sl]
            kkey = a // KV
            kid = a - kkey * KV
            idx1_c[sl] = o * RPAD + kkey
            idx2_c[sl] = o * 128 + kid

    def fire_g(idx1_c, idx2_c, s1_c, s2_c, gsem):
        pltpu.async_copy(sf.at[idx1_c], s1_c, gsem)
        pltpu.async_copy(qpf.at[idx2_c], s2_c, gsem)

    def finish(ci, kq1_c, idx1_c, idx2_c, s1_c, s2_c, gsem):
        pltpu.make_async_copy(sf.at[idx1_c], s1_c, gsem).wait()
        pltpu.make_async_copy(qpf.at[idx2_c], s2_c, gsem).wait()
        off = ci * CH
        for g in range(CH // 16):
            sl = pl.ds(g * 16, 16)
            ev = jnp.exp((s1_c[sl] + s2_c[sl]) * SCALE)
            ev = jnp.where(base0 + off + g * 16 + iota16 < MREAL, ev, 0.0)
            e_str[pl.ds(off + g * 16, 16)] = ev
            plsc.addupdate_scatter(den_v, [kq1_c[sl]], ev)

    fire_kq(0, kq0_a, kq1_a, lsem_a)
    lax.fori_loop(0, NV // 16, zbody, 0)
    wait_kq(kq0_a, kq1_a, lsem_a)
    prep(kq0_a, kq1_a, idx1_a, idx2_a)
    fire_g(idx1_a, idx2_a, s1_a, s2_a, gsem_a)
    fire_kq(1, kq0_b, kq1_b, lsem_b)

    def pipe(ci2, _):
        o = 2 * ci2 + 1
        wait_kq(kq0_b, kq1_b, lsem_b)
        prep(kq0_b, kq1_b, idx1_b, idx2_b)
        fire_g(idx1_b, idx2_b, s1_b, s2_b, gsem_b)

        @pl.when(o + 1 < NCH2)
        def _():
            fire_kq(o + 1, kq0_a, kq1_a, lsem_a)

        finish(2 * ci2, kq1_a, idx1_a, idx2_a, s1_a, s2_a, gsem_a)

        @pl.when(o + 1 < NCH2)
        def _():
            wait_kq(kq0_a, kq1_a, lsem_a)
            prep(kq0_a, kq1_a, idx1_a, idx2_a)
            fire_g(idx1_a, idx2_a, s1_a, s2_a, gsem_a)
            fire_kq(o + 2, kq0_b, kq1_b, lsem_b)

        finish(o, kq1_b, idx1_b, idx2_b, s1_b, s2_b, gsem_b)
        return 0

    lax.fori_loop(0, NCH2 // 2, pipe, 0)
    pltpu.sync_copy(e_str, e_out.at[pl.ds(base0, TP)])
    pltpu.sync_copy(den_v, dpart.at[wid])


def _k2(sf, qpf, kq0, kq1):
    mesh = plsc.VectorSubcoreMesh(core_axis_name="c", subcore_axis_name="s")
    buf = [
        pltpu.VMEM((CH,), jnp.int32),
        pltpu.VMEM((CH,), jnp.int32),
        pltpu.VMEM((CH,), jnp.int32),
        pltpu.VMEM((CH,), jnp.int32),
        pltpu.VMEM((CH,), jnp.float32),
        pltpu.VMEM((CH,), jnp.float32),
    ]
    f = pl.kernel(
        _k2_body,
        compiler_params=pltpu.CompilerParams(needs_layout_passes=False),
        out_type=[
            jax.ShapeDtypeStruct((MP,), jnp.float32),
            jax.ShapeDtypeStruct((32, NV), jnp.float32),
        ],
        mesh=mesh,
        scratch_types=(
            [pltpu.VMEM((NV,), jnp.float32), pltpu.VMEM((TP,), jnp.float32)]
            + buf + buf
            + [
                pltpu.SemaphoreType.DMA,
                pltpu.SemaphoreType.DMA,
                pltpu.SemaphoreType.DMA,
                pltpu.SemaphoreType.DMA,
            ]
        ),
    )
    return f(sf, qpf, kq0, kq1)


def _k3_body(dp_ref, o_ref):
    o_ref[...] = 1.0 / jnp.sum(dp_ref[...], axis=0, keepdims=True)


def _k3(dpart):
    return pl.pallas_call(
        _k3_body,
        out_shape=jax.ShapeDtypeStruct((1, NV), jnp.float32),
    )(dpart)


TPW = MP // 16
NSTR = 624
REM = NV - 16 * NSTR
NCH4 = TPW // CH


def _k4_body(vcat, e_in, invd, kq0, kq1, zer, out_hbm,
             invd_v,
             kq0_a, kq1_a, e_a, vidx_a, oidx_a, attn_a, rows_a,
             kq0_b, kq1_b, e_b, vidx_b, oidx_b, attn_b, rows_b,
             out_sp, sem_a, sem_b, lsem_a, lsem_b):
    c = lax.axis_index("c")
    s = lax.axis_index("s")
    base0 = s * TPW
    coff = c * NV

    def fire_kq(ci, kq0_c, kq1_c, e_c, lsem):
        base = base0 + ci * CH
        pltpu.async_copy(kq0.at[pl.ds(base, CH)], kq0_c, lsem)
        pltpu.async_copy(kq1.at[pl.ds(base, CH)], kq1_c, lsem)
        pltpu.async_copy(e_in.at[pl.ds(base, CH)], e_c, lsem)

    def wait_kq(kq0_c, kq1_c, e_c, lsem):
        pltpu.make_async_copy(kq0.at[pl.ds(0, CH)], kq0_c, lsem).wait()
        pltpu.make_async_copy(kq1.at[pl.ds(0, CH)], kq1_c, lsem).wait()
        pltpu.make_async_copy(e_in.at[pl.ds(0, CH)], e_c, lsem).wait()

    def prep(kq0_c, kq1_c, e_c, vidx_buf, oidx_buf, attn_buf):
        for g in range(CH // 16):
            sl = pl.ds(g * 16, 16)
            a = kq0_c[sl]
            o = kq1_c[sl]
            vidx_buf[sl] = a // KV + coff
            oidx_buf[sl] = o
            d = plsc.load_gather(invd_v, [o])
            attn_buf[sl] = e_c[sl] * d

    def work(rows_buf, oidx_buf, attn_buf):
        def scale(p, _):
            a16 = plsc.load_gather(attn_buf, [jnp.zeros((16,), jnp.int32) + p])
            for j in range(8):
                rows_buf[p, pl.ds(j * 16, 16)] = (
                    rows_buf[p, pl.ds(j * 16, 16)] * a16)
            return 0

        lax.fori_loop(0, CH, scale, 0, unroll=2)
        pltpu.sync_copy(rows_buf, out_sp.at[oidx_buf], add=True)

    fire_kq(0, kq0_a, kq1_a, e_a, lsem_a)
    cpi = pltpu.async_copy(invd, invd_v, sem_a)
    pltpu.sync_copy(zer, out_sp.at[pl.ds(s * NSTR, NSTR)])

    @pl.when(s == 0)
    def _():
        pltpu.sync_copy(zer.at[pl.ds(0, REM)],
                        out_sp.at[pl.ds(16 * NSTR, REM)])

    cpi.wait()
    plsc.subcore_barrier()
    wait_kq(kq0_a, kq1_a, e_a, lsem_a)
    prep(kq0_a, kq1_a, e_a, vidx_a, oidx_a, attn_a)
    pltpu.async_copy(vcat.at[vidx_a], rows_a, sem_a)
    fire_kq(1, kq0_b, kq1_b, e_b, lsem_b)

    def pipe(ci2, _):
        o = 2 * ci2 + 1
        wait_kq(kq0_b, kq1_b, e_b, lsem_b)
        prep(kq0_b, kq1_b, e_b, vidx_b, oidx_b, attn_b)
        pltpu.async_copy(vcat.at[vidx_b], rows_b, sem_b)

        @pl.when(o + 1 < NCH4)
        def _():
            fire_kq(o + 1, kq0_a, kq1_a, e_a, lsem_a)

        pltpu.make_async_copy(vcat.at[vidx_a], rows_a, sem_a).wait()
        work(rows_a, oidx_a, attn_a)

        @pl.when(o + 1 < NCH4)
        def _():
            wait_kq(kq0_a, kq1_a, e_a, lsem_a)
            prep(kq0_a, kq1_a, e_a, vidx_a, oidx_a, attn_a)
            pltpu.async_copy(vcat.at[vidx_a], rows_a, sem_a)
            fire_kq(o + 2, kq0_b, kq1_b, e_b, lsem_b)

        pltpu.make_async_copy(vcat.at[vidx_b], rows_b, sem_b).wait()
        work(rows_b, oidx_b, attn_b)
        return 0

    lax.fori_loop(0, NCH4 // 2, pipe, 0)
    plsc.subcore_barrier()
    pltpu.sync_copy(out_sp.at[pl.ds(s * NSTR, NSTR)],
                    out_hbm.at[pl.ds(s * NSTR, NSTR), pl.ds(c * 128, 128)])

    @pl.when(s == 0)
    def _():
        pltpu.sync_copy(out_sp.at[pl.ds(16 * NSTR, REM)],
                        out_hbm.at[pl.ds(16 * NSTR, REM), pl.ds(c * 128, 128)])


def _k4(vcat, e, invd, kq0, kq1, zer):
    mesh = plsc.VectorSubcoreMesh(core_axis_name="c", subcore_axis_name="s")
    buf = [
        pltpu.VMEM((CH,), jnp.int32),
        pltpu.VMEM((CH,), jnp.int32),
        pltpu.VMEM((CH,), jnp.float32),
        pltpu.VMEM((CH,), jnp.int32),
        pltpu.VMEM((CH,), jnp.int32),
        pltpu.VMEM((CH,), jnp.float32),
        pltpu.VMEM((CH, 128), jnp.float32),
    ]
    f = pl.kernel(
        _k4_body,
        compiler_params=pltpu.CompilerParams(needs_layout_passes=False),
        out_type=jax.ShapeDtypeStruct((NV, 256), jnp.float32),
        mesh=mesh,
        scratch_types=(
            [pltpu.VMEM((NV,), jnp.float32)] + buf + buf
            + [
                pltpu.VMEM_SHARED((NV, 128), jnp.float32),
                pltpu.SemaphoreType.DMA,
                pltpu.SemaphoreType.DMA,
                pltpu.SemaphoreType.DMA,
                pltpu.SemaphoreType.DMA,
            ]
        ),
    )
    return f(vcat, e, invd, kq0, kq1, zer)


def kernel(q, k, v, pos_enc, kq_map):
    S1f = _s1(q, k).reshape(-1)
    Qpf = _qp(q, pos_enc).reshape(-1)

    pad = MP - kq_map.shape[1]
    kq0 = jnp.concatenate([kq_map[0], jnp.zeros((pad,), kq_map.dtype)])
    kq1 = jnp.concatenate([kq_map[1], jnp.zeros((pad,), kq_map.dtype)])

    e, dpart = _k2(S1f, Qpf, kq0, kq1)
    invd = _k3(dpart).reshape(-1)

    vcat = _vcat(v).reshape(2 * NV, 128)
    zer = jnp.zeros((NSTR, 128), jnp.float32)
    return _k4(vcat, e, invd, kq0, kq1, zer)


# speedup vs baseline: 12.9740x; 1.0004x over previous
import functools

import jax
import jax.numpy as jnp
from jax import lax
from jax.experimental import pallas as pl
from jax.experimental.pallas import tpu as pltpu
from jax.experimental.pallas import tpu_sc as plsc

NV = 10000
KV = 27
RPAD = 10240
MREAL = NV * KV
MP = 270336
TP = MP // 32
CH = 128
SCALE = 1.0 / 16.0


def _s1_body(a_ref, k_ref, o_ref):
    res = jax.lax.dot_general(
        a_ref[...].astype(jnp.bfloat16), k_ref[...].astype(jnp.bfloat16),
        (((1,), (1,)), ((), ())),
        preferred_element_type=jnp.float32)
    res = jnp.concatenate(
        [res, jnp.zeros((res.shape[0], RPAD - NV), jnp.float32)], axis=1)
    o_ref[...] = res.reshape(res.shape[0], RPAD // 128, 128)


def _s1(q, k):
    BM = 400
    return pl.pallas_call(
        _s1_body,
        grid=(NV // BM,),
        in_specs=[
            pl.BlockSpec((BM, 256), lambda i: (i, 0)),
            pl.BlockSpec((NV, 256), lambda i: (0, 0)),
        ],
        out_specs=pl.BlockSpec((BM, RPAD // 128, 128), lambda i: (i, 0, 0)),
        out_shape=jax.ShapeDtypeStruct((NV, RPAD // 128, 128), jnp.float32),
    )(q, k)


def _qp_body(a_ref, p_ref, o_ref):
    res = jax.lax.dot_general(
        a_ref[...], p_ref[...], (((1,), (1,)), ((), ())),
        preferred_element_type=jnp.float32)
    o_ref[...] = jnp.concatenate(
        [res, jnp.zeros((res.shape[0], 128 - KV), jnp.float32)], axis=1)


def _qp(q, pos_enc):
    BM = 2000
    return pl.pallas_call(
        _qp_body,
        grid=(NV // BM,),
        in_specs=[
            pl.BlockSpec((BM, 256), lambda i: (i, 0)),
            pl.BlockSpec((KV, 256), lambda i: (0, 0)),
        ],
        out_specs=pl.BlockSpec((BM, 128), lambda i: (i, 0)),
        out_shape=jax.ShapeDtypeStruct((NV, 128), jnp.float32),
    )(q, pos_enc)


def _vcat_body(v_ref, o_ref):
    o_ref[...] = v_ref[...][None]


def _vcat(v):
    return pl.pallas_call(
        _vcat_body,
        grid=(2,),
        in_specs=[pl.BlockSpec((NV, 128), lambda c: (0, c))],
        out_specs=pl.BlockSpec((1, NV, 128), lambda c: (c, 0, 0)),
        out_shape=jax.ShapeDtypeStruct((2, NV, 128), jnp.float32),
    )(v)


NCH2 = TP // CH


def _k2_body(sf, qpf, kq0, kq1, e_out, dpart,
             den_v, e_str,
             kq0_a, kq1_a, oidx_a, idx1_a, idx2_a, s1_a, s2_a,
             kq0_b, kq1_b, oidx_b, idx1_b, idx2_b, s1_b, s2_b,
             gsem_a, gsem_b, lsem_a, lsem_b):
    c = lax.axis_index("c")
    s = lax.axis_index("s")
    wid = s * 2 + c
    base0 = wid * TP
    iota16 = jax.lax.iota(jnp.int32, 16)
    zero16 = jnp.zeros((16,), jnp.float32)

    def zbody(i, _):
        den_v[pl.ds(i * 16, 16)] = zero16
        return 0

    def fire_kq(ci, kq0_c, kq1_c, lsem):
        base = base0 + ci * CH
        pltpu.async_copy(kq0.at[pl.ds(base, CH)], kq0_c, lsem)
        pltpu.async_copy(kq1.at[pl.ds(base, CH)], kq1_c, lsem)

    def wait_kq(kq0_c, kq1_c, lsem):
        pltpu.make_async_copy(kq0.at[pl.ds(0, CH)], kq0_c, lsem).wait()
        pltpu.make_async_copy(kq1.at[pl.ds(0, CH)], kq1_c, lsem).wait()

    def prep(kq0_c, kq1_c, oidx_c, idx1_c, idx2_c):
        for g in range(CH // 16):
            sl = pl.ds(g * 16, 16)
            a = kq0_c[sl]
            o = kq1_c[sl]
            kkey = a // KV
            kid = a - kkey * KV
            oidx_c[sl] = o
            idx1_c[sl] = o * RPAD + kkey
            idx2_c[sl] = o * 128 + kid

    def fire_g(idx1_c, idx2_c, s1_c, s2_c, gsem):
        pltpu.async_copy(sf.at[idx1_c], s1_c, gsem)
        pltpu.async_copy(qpf.at[idx2_c], s2_c, gsem)

    def finish(ci, oidx_c, idx1_c, idx2_c, s1_c, s2_c, gsem):
        pltpu.make_async_copy(sf.at[idx1_c], s1_c, gsem).wait()
        pltpu.make_async_copy(qpf.at[idx2_c], s2_c, gsem).wait()
        off = ci * CH
        for g in range(CH // 16):
            sl = pl.ds(g * 16, 16)
            ev = jnp.exp((s1_c[sl] + s2_c[sl]) * SCALE)
            ev = jnp.where(base0 + off + g * 16 + iota16 < MREAL, ev, 0.0)
            e_str[pl.ds(off + g * 16, 16)] = ev
            plsc.addupdate_scatter(den_v, [oidx_c[sl]], ev)

    fire_kq(0, kq0_a, kq1_a, lsem_a)
    lax.fori_loop(0, NV // 16, zbody, 0)
    wait_kq(kq0_a, kq1_a, lsem_a)
    prep(kq0_a, kq1_a, oidx_a, idx1_a, idx2_a)
    fire_g(idx1_a, idx2_a, s1_a, s2_a, gsem_a)
    fire_kq(1, kq0_b, kq1_b, lsem_b)

    def pipe(ci2, _):
        o = 2 * ci2 + 1
        wait_kq(kq0_b, kq1_b, lsem_b)
        prep(kq0_b, kq1_b, oidx_b, idx1_b, idx2_b)
        fire_g(idx1_b, idx2_b, s1_b, s2_b, gsem_b)

        @pl.when(o + 1 < NCH2)
        def _():
            fire_kq(o + 1, kq0_a, kq1_a, lsem_a)

        finish(2 * ci2, oidx_a, idx1_a, idx2_a, s1_a, s2_a, gsem_a)

        @pl.when(o + 1 < NCH2)
        def _():
            wait_kq(kq0_a, kq1_a, lsem_a)
            prep(kq0_a, kq1_a, oidx_a, idx1_a, idx2_a)
            fire_g(idx1_a, idx2_a, s1_a, s2_a, gsem_a)
            fire_kq(o + 2, kq0_b, kq1_b, lsem_b)

        finish(o, oidx_b, idx1_b, idx2_b, s1_b, s2_b, gsem_b)
        return 0

    lax.fori_loop(0, NCH2 // 2, pipe, 0)
    pltpu.sync_copy(e_str, e_out.at[pl.ds(base0, TP)])
    pltpu.sync_copy(den_v, dpart.at[wid])


def _k2(sf, qpf, kq0, kq1):
    mesh = plsc.VectorSubcoreMesh(core_axis_name="c", subcore_axis_name="s")
    buf = [
        pltpu.VMEM((CH,), jnp.int32),
        pltpu.VMEM((CH,), jnp.int32),
        pltpu.VMEM((CH,), jnp.int32),
        pltpu.VMEM((CH,), jnp.int32),
        pltpu.VMEM((CH,), jnp.int32),
        pltpu.VMEM((CH,), jnp.float32),
        pltpu.VMEM((CH,), jnp.float32),
    ]
    f = pl.kernel(
        _k2_body,
        compiler_params=pltpu.CompilerParams(needs_layout_passes=False),
        out_type=[
            jax.ShapeDtypeStruct((MP,), jnp.float32),
            jax.ShapeDtypeStruct((32, NV), jnp.float32),
        ],
        mesh=mesh,
        scratch_types=(
            [pltpu.VMEM((NV,), jnp.float32), pltpu.VMEM((TP,), jnp.float32)]
            + buf + buf
            + [
                pltpu.SemaphoreType.DMA,
                pltpu.SemaphoreType.DMA,
                pltpu.SemaphoreType.DMA,
                pltpu.SemaphoreType.DMA,
            ]
        ),
    )
    return f(sf, qpf, kq0, kq1)


def _k3_body(dp_ref, o_ref):
    o_ref[...] = 1.0 / jnp.sum(dp_ref[...], axis=0, keepdims=True)


def _k3(dpart):
    return pl.pallas_call(
        _k3_body,
        out_shape=jax.ShapeDtypeStruct((1, NV), jnp.float32),
    )(dpart)


TPW = MP // 16
NSTR = 624
REM = NV - 16 * NSTR
NCH4 = TPW // CH


def _k4_body(vcat, e_in, invd, kq0, kq1, zer, out_hbm,
             invd_v,
             kq0_a, kq1_a, e_a, vidx_a, oidx_a, attn_a, rows_a,
             kq0_b, kq1_b, e_b, vidx_b, oidx_b, attn_b, rows_b,
             out_sp, sem_a, sem_b, lsem_a, lsem_b):
    c = lax.axis_index("c")
    s = lax.axis_index("s")
    base0 = s * TPW
    coff = c * NV

    def fire_kq(ci, kq0_c, kq1_c, e_c, lsem):
        base = base0 + ci * CH
        pltpu.async_copy(kq0.at[pl.ds(base, CH)], kq0_c, lsem)
        pltpu.async_copy(kq1.at[pl.ds(base, CH)], kq1_c, lsem)
        pltpu.async_copy(e_in.at[pl.ds(base, CH)], e_c, lsem)

    def wait_kq(kq0_c, kq1_c, e_c, lsem):
        pltpu.make_async_copy(kq0.at[pl.ds(0, CH)], kq0_c, lsem).wait()
        pltpu.make_async_copy(kq1.at[pl.ds(0, CH)], kq1_c, lsem).wait()
        pltpu.make_async_copy(e_in.at[pl.ds(0, CH)], e_c, lsem).wait()

    def prep(kq0_c, kq1_c, e_c, vidx_buf, oidx_buf, attn_buf):
        for g in range(CH // 16):
            sl = pl.ds(g * 16, 16)
            a = kq0_c[sl]
            o = kq1_c[sl]
            vidx_buf[sl] = a // KV + coff
            oidx_buf[sl] = o
            d = plsc.load_gather(invd_v, [o])
            attn_buf[sl] = e_c[sl] * d

    def work(rows_buf, oidx_buf, attn_buf):
        def scale(p, _):
            a16 = plsc.load_gather(attn_buf, [jnp.zeros((16,), jnp.int32) + p])
            for j in range(8):
                rows_buf[p, pl.ds(j * 16, 16)] = (
                    rows_buf[p, pl.ds(j * 16, 16)] * a16)
            return 0

        lax.fori_loop(0, CH, scale, 0, unroll=2)
        pltpu.sync_copy(rows_buf, out_sp.at[oidx_buf], add=True)

    fire_kq(0, kq0_a, kq1_a, e_a, lsem_a)
    cpi = pltpu.async_copy(invd, invd_v, sem_a)
    pltpu.sync_copy(zer, out_sp.at[pl.ds(s * NSTR, NSTR)])

    @pl.when(s == 0)
    def _():
        pltpu.sync_copy(zer.at[pl.ds(0, REM)],
                        out_sp.at[pl.ds(16 * NSTR, REM)])

    cpi.wait()
    plsc.subcore_barrier()
    wait_kq(kq0_a, kq1_a, e_a, lsem_a)
    prep(kq0_a, kq1_a, e_a, vidx_a, oidx_a, attn_a)
    pltpu.async_copy(vcat.at[vidx_a], rows_a, sem_a)
    fire_kq(1, kq0_b, kq1_b, e_b, lsem_b)

    def pipe(ci2, _):
        o = 2 * ci2 + 1
        wait_kq(kq0_b, kq1_b, e_b, lsem_b)
        prep(kq0_b, kq1_b, e_b, vidx_b, oidx_b, attn_b)
        pltpu.async_copy(vcat.at[vidx_b], rows_b, sem_b)

        @pl.when(o + 1 < NCH4)
        def _():
            fire_kq(o + 1, kq0_a, kq1_a, e_a, lsem_a)

        pltpu.make_async_copy(vcat.at[vidx_a], rows_a, sem_a).wait()
        work(rows_a, oidx_a, attn_a)

        @pl.when(o + 1 < NCH4)
        def _():
            wait_kq(kq0_a, kq1_a, e_a, lsem_a)
            prep(kq0_a, kq1_a, e_a, vidx_a, oidx_a, attn_a)
            pltpu.async_copy(vcat.at[vidx_a], rows_a, sem_a)
            fire_kq(o + 2, kq0_b, kq1_b, e_b, lsem_b)

        pltpu.make_async_copy(vcat.at[vidx_b], rows_b, sem_b).wait()
        work(rows_b, oidx_b, attn_b)
        return 0

    lax.fori_loop(0, NCH4 // 2, pipe, 0)
    plsc.subcore_barrier()
    pltpu.sync_copy(out_sp.at[pl.ds(s * NSTR, NSTR)],
                    out_hbm.at[pl.ds(s * NSTR, NSTR), pl.ds(c * 128, 128)])

    @pl.when(s == 0)
    def _():
        pltpu.sync_copy(out_sp.at[pl.ds(16 * NSTR, REM)],
                        out_hbm.at[pl.ds(16 * NSTR, REM), pl.ds(c * 128, 128)])


def _k4(vcat, e, invd, kq0, kq1, zer):
    mesh = plsc.VectorSubcoreMesh(core_axis_name="c", subcore_axis_name="s")
    buf = [
        pltpu.VMEM((CH,), jnp.int32),
        pltpu.VMEM((CH,), jnp.int32),
        pltpu.VMEM((CH,), jnp.float32),
        pltpu.VMEM((CH,), jnp.int32),
        pltpu.VMEM((CH,), jnp.int32),
        pltpu.VMEM((CH,), jnp.float32),
        pltpu.VMEM((CH, 128), jnp.float32),
    ]
    f = pl.kernel(
        _k4_body,
        compiler_params=pltpu.CompilerParams(needs_layout_passes=False),
        out_type=jax.ShapeDtypeStruct((NV, 256), jnp.float32),
        mesh=mesh,
        scratch_types=(
            [pltpu.VMEM((NV,), jnp.float32)] + buf + buf
            + [
                pltpu.VMEM_SHARED((NV, 128), jnp.float32),
                pltpu.SemaphoreType.DMA,
                pltpu.SemaphoreType.DMA,
                pltpu.SemaphoreType.DMA,
                pltpu.SemaphoreType.DMA,
            ]
        ),
    )
    return f(vcat, e, invd, kq0, kq1, zer)


def kernel(q, k, v, pos_enc, kq_map):
    S1f = _s1(q, k).reshape(-1)
    Qpf = _qp(q, pos_enc).reshape(-1)

    pad = MP - kq_map.shape[1]
    kq0 = jnp.concatenate([kq_map[0], jnp.zeros((pad,), kq_map.dtype)])
    kq1 = jnp.concatenate([kq_map[1], jnp.zeros((pad,), kq_map.dtype)])

    e, dpart = _k2(S1f, Qpf, kq0, kq1)
    invd = _k3(dpart).reshape(-1)

    vcat = _vcat(v).reshape(2 * NV, 128)
    zer = jnp.zeros((NSTR, 128), jnp.float32)
    return _k4(vcat, e, invd, kq0, kq1, zer)
